# baseline (speedup 1.0000x reference)
import functools

import jax
import jax.numpy as jnp
from jax import lax
from jax.experimental import pallas as pl
from jax.experimental.pallas import tpu as pltpu

_BF16 = jnp.bfloat16
_EPS = 1e-5
_VMEM = 48 * 1024 * 1024


def _cp():
    return pltpu.CompilerParams(dimension_semantics=("parallel",),
                                vmem_limit_bytes=_VMEM)


def _r8(n):
    return (n + 7) // 8 * 8


def _tap_weights_merged(w):
    c_out, c_in = w.shape[0], w.shape[1]
    wt = jnp.transpose(w, (2, 3, 1, 0)).astype(_BF16)
    taps = [wt[2 * dh:2 * dh + 2, 2 * dw:2 * dw + 2].reshape(4 * c_in, c_out)
            for dh in (0, 1) for dw in (0, 1)]
    return jnp.stack(taps)


def _tap_weights_split(w):
    c_out, c_in = w.shape[0], w.shape[1]
    wt = jnp.transpose(w, (2, 3, 1, 0)).astype(_BF16)
    taps = [wt[2 * dh + p, 2 * dw:2 * dw + 2].reshape(2 * c_in, c_out)
            for p in (0, 1) for dh in (0, 1) for dw in (0, 1)]
    return jnp.stack(taps)


def _stat_mask(mo, mw, ow, c):
    valid = (jnp.arange(mo, dtype=jnp.int32) % mw) < ow
    return jnp.broadcast_to(valid[:, None], (mo, c)).astype(jnp.float32)


def _norm_leaky(h, b_ref, m_ref, nvalid):
    h = h + b_ref[...]
    hm = h * m_ref[...]
    inv_n = 1.0 / nvalid
    mu = jnp.sum(hm, axis=0, keepdims=True) * inv_n
    var = jnp.sum(hm * hm, axis=0, keepdims=True) * inv_n - mu * mu
    h = (h - mu) * lax.rsqrt(var + _EPS)
    return jnp.maximum(h, 0.2 * h)


def _pair_merge(x, rows, ow, c):
    r = x.reshape(rows, ow // 2, 2, c)
    s0 = r[:, :, 0, :]
    s1 = r[:, :, 1, :]
    z = jnp.zeros((rows, 1, c), x.dtype)
    return jnp.concatenate([jnp.concatenate([z, s1], axis=1),
                            jnp.concatenate([s0, z], axis=1)], axis=-1)


def _store_parity_merged(hb, oe_ref, oo_ref, oh, mw, ow, c, w2m):
    v4 = hb.reshape(oh // 2, 2, mw, c)[:, :, :ow, :]
    pe = _pair_merge(v4[:, 1], oh // 2, ow, c)
    po = _pair_merge(v4[:, 0], oh // 2, ow, c)
    oe_ref[0] = jnp.zeros((oh // 2 + 2, w2m, 2 * c), _BF16)
    oo_ref[0] = jnp.zeros((oh // 2 + 2, w2m, 2 * c), _BF16)
    oe_ref[0, 1:oh // 2 + 1, 0:ow // 2 + 1, :] = pe
    oo_ref[0, 0:oh // 2, 0:ow // 2 + 1, :] = po


def _l1_body(p_ref, w_ref, b_ref, e0_ref, e1_ref, o0_ref, o1_ref,
             *, oh, ow, c, w2m):
    h = jnp.dot(p_ref[0], w_ref[...], preferred_element_type=jnp.float32)
    h = h + b_ref[...]
    h = jnp.maximum(h, 0.2 * h)
    hb = h.astype(_BF16)
    v = hb.reshape(oh // 2, 2, ow // 2, 2, c)
    hf = oh // 2
    wf = ow // 2
    z = jnp.zeros((hf + 2, w2m, c), _BF16)
    e0_ref[0] = z
    e1_ref[0] = z
    o0_ref[0] = z
    o1_ref[0] = z
    e0_ref[0, 1:hf + 1, 1:wf + 1, :] = v[:, 1, :, 1, :]
    e1_ref[0, 1:hf + 1, 0:wf, :] = v[:, 1, :, 0, :]
    o0_ref[0, 0:hf, 1:wf + 1, :] = v[:, 0, :, 1, :]
    o1_ref[0, 0:hf, 0:wf, :] = v[:, 0, :, 0, :]


def _layer1(x, w1, b1):
    B, H, W, C = x.shape
    oh, ow = H // 2, W // 2
    c_out = w1.shape[0]
    xp = jnp.pad(x, ((0, 0), (1, 1), (1, 1), (0, 0)))
    s = xp.reshape(B, oh + 1, 2, ow + 1, 2, C)
    s = s.transpose(0, 1, 3, 2, 4, 5).reshape(B, oh + 1, ow + 1, 4 * C)
    p = jnp.concatenate(
        [s[:, dh:dh + oh, dw:dw + ow] for dh in (0, 1) for dw in (0, 1)],
        axis=-1).reshape(B, oh * ow, 16 * C)
    wt = jnp.transpose(w1, (2, 3, 1, 0)).astype(_BF16)
    w2d = wt.reshape(2, 2, 2, 2, C, c_out).transpose(0, 2, 1, 3, 4, 5)
    w2d = w2d.reshape(16 * C, c_out)
    rr = oh // 2 + 2
    w2m = _r8(ow // 2 + 1)
    body = functools.partial(_l1_body, oh=oh, ow=ow, c=c_out, w2m=w2m)
    osd = jax.ShapeDtypeStruct((B, rr, w2m, c_out), _BF16)
    obs = pl.BlockSpec((1, rr, w2m, c_out), lambda b: (b, 0, 0, 0))
    e0, e1, o0, o1 = pl.pallas_call(
        body,
        out_shape=(osd, osd, osd, osd),
        grid=(B,),
        in_specs=[
            pl.BlockSpec((1, oh * ow, 16 * C), lambda b: (b, 0, 0)),
            pl.BlockSpec((16 * C, c_out), lambda b: (0, 0)),
            pl.BlockSpec((1, c_out), lambda b: (0, 0)),
        ],
        out_specs=(obs, obs, obs, obs),
        compiler_params=_cp(),
    )(p, w2d, b1.reshape(1, c_out).astype(jnp.float32))
    eo = jnp.concatenate([e0, e1, o0, o1], axis=-1)
    return eo.reshape(B, rr * w2m, 4 * c_out), w2m


def _l2_body(eo_ref, w_ref, b_ref, m_ref, oe_ref, oo_ref, *, mw, mo, oh, ow,
             nvalid, c, w2m):
    shifts = (0, 1, mw, mw + 1)
    h = jnp.dot(eo_ref[0, pl.ds(0, mo), :], w_ref[0],
                preferred_element_type=jnp.float32)
    for t in range(1, 4):
        h = h + jnp.dot(eo_ref[0, pl.ds(shifts[t], mo), :], w_ref[t],
                        preferred_element_type=jnp.float32)
    h = _norm_leaky(h, b_ref, m_ref, nvalid)
    _store_parity_merged(h.astype(_BF16), oe_ref, oo_ref, oh, mw, ow, c, w2m)


def _layer2(eo_flat, w, b, oh, ow, mw):
    B, L, k4 = eo_flat.shape
    c_out = w.shape[0]
    mo = oh * mw
    w_taps = _tap_weights_merged(w)
    mask = _stat_mask(mo, mw, ow, c_out)
    rr = oh // 2 + 2
    w2m = _r8(ow // 2 + 1)
    body = functools.partial(_l2_body, mw=mw, mo=mo, oh=oh, ow=ow,
                             nvalid=oh * ow, c=c_out, w2m=w2m)
    osd = jax.ShapeDtypeStruct((B, rr, w2m, 2 * c_out), _BF16)
    obs = pl.BlockSpec((1, rr, w2m, 2 * c_out), lambda b: (b, 0, 0, 0))
    e, o = pl.pallas_call(
        body,
        out_shape=(osd, osd),
        grid=(B,),
        in_specs=[
            pl.BlockSpec((1, L, k4), lambda b: (b, 0, 0)),
            pl.BlockSpec((4, k4, c_out), lambda b: (0, 0, 0)),
            pl.BlockSpec((1, c_out), lambda b: (0, 0)),
            pl.BlockSpec((mo, c_out), lambda b: (0, 0)),
        ],
        out_specs=(obs, obs),
        compiler_params=_cp(),
    )(eo_flat, w_taps, b.reshape(1, c_out).astype(jnp.float32), mask)
    return (e.reshape(B, rr * w2m, 2 * c_out),
            o.reshape(B, rr * w2m, 2 * c_out), w2m)


def _l3_body(e_ref, o_ref, w_ref, b_ref, m_ref, oe_ref, oo_ref, *, mw, mo,
             oh, ow, nvalid, c, w2m):
    h = None
    for i, (dh, dw) in enumerate(((0, 0), (0, 1), (1, 0), (1, 1))):
        s = dh * mw + dw
        d = jnp.dot(e_ref[0, pl.ds(s, mo), :], w_ref[i],
                    preferred_element_type=jnp.float32)
        h = d if h is None else h + d
        h = h + jnp.dot(o_ref[0, pl.ds(s, mo), :], w_ref[4 + i],
                        preferred_element_type=jnp.float32)
    h = _norm_leaky(h, b_ref, m_ref, nvalid)
    _store_parity_merged(h.astype(_BF16), oe_ref, oo_ref, oh, mw, ow, c, w2m)


def _layer3(e_flat, o_flat, w, b, oh, ow, mw):
    B, L, k2 = e_flat.shape
    c_out = w.shape[0]
    mo = oh * mw
    w_taps = _tap_weights_split(w)
    mask = _stat_mask(mo, mw, ow, c_out)
    rr = oh // 2 + 2
    w2m = _r8(ow // 2 + 1)
    body = functools.partial(_l3_body, mw=mw, mo=mo, oh=oh, ow=ow,
                             nvalid=oh * ow, c=c_out, w2m=w2m)
    osd = jax.ShapeDtypeStruct((B, rr, w2m, 2 * c_out), _BF16)
    obs = pl.BlockSpec((1, rr, w2m, 2 * c_out), lambda b: (b, 0, 0, 0))
    ibs = pl.BlockSpec((1, L, k2), lambda b: (b, 0, 0))
    e, o = pl.pallas_call(
        body,
        out_shape=(osd, osd),
        grid=(B,),
        in_specs=[
            ibs, ibs,
            pl.BlockSpec((8, k2, c_out), lambda b: (0, 0, 0)),
            pl.BlockSpec((1, c_out), lambda b: (0, 0)),
            pl.BlockSpec((mo, c_out), lambda b: (0, 0)),
        ],
        out_specs=(obs, obs),
        compiler_params=_cp(),
    )(e_flat, o_flat, w_taps, b.reshape(1, c_out).astype(jnp.float32), mask)
    return (e.reshape(B, rr * w2m, 2 * c_out),
            o.reshape(B, rr * w2m, 2 * c_out), w2m)


def _l4_body(e_ref, o_ref, w_ref, b_ref, m_ref, o5_ref, *, mw, mo, oh, ow,
             nvalid, c, wp):
    h = None
    for i, (dh, dw) in enumerate(((0, 0), (0, 1), (1, 0), (1, 1))):
        s = dh * mw + dw
        d = jnp.dot(e_ref[0, pl.ds(s, mo), :], w_ref[i],
                    preferred_element_type=jnp.float32)
        h = d if h is None else h + d
        h = h + jnp.dot(o_ref[0, pl.ds(s, mo), :], w_ref[4 + i],
                        preferred_element_type=jnp.float32)
    h = _norm_leaky(h, b_ref, m_ref, nvalid)
    hb = h.astype(_BF16).reshape(oh, mw, c)[:, :ow, :]
    o5_ref[0] = jnp.zeros((oh + 4, wp, c), _BF16)
    o5_ref[0, 2:oh + 2, 2:ow + 2, :] = hb


def _layer4(e_flat, o_flat, w, b, oh, ow, mw):
    B, L, k2 = e_flat.shape
    c_out = w.shape[0]
    mo = oh * mw
    w_taps = _tap_weights_split(w)
    mask = _stat_mask(mo, mw, ow, c_out)
    wp = _r8(ow + 3)
    body = functools.partial(_l4_body, mw=mw, mo=mo, oh=oh, ow=ow,
                             nvalid=oh * ow, c=c_out, wp=wp)
    osd = jax.ShapeDtypeStruct((B, oh + 4, wp, c_out), _BF16)
    obs = pl.BlockSpec((1, oh + 4, wp, c_out), lambda b: (b, 0, 0, 0))
    ibs = pl.BlockSpec((1, L, k2), lambda b: (b, 0, 0))
    out = pl.pallas_call(
        body,
        out_shape=osd,
        grid=(B,),
        in_specs=[
            ibs, ibs,
            pl.BlockSpec((8, k2, c_out), lambda b: (0, 0, 0)),
            pl.BlockSpec((1, c_out), lambda b: (0, 0)),
            pl.BlockSpec((mo, c_out), lambda b: (0, 0)),
        ],
        out_specs=obs,
        compiler_params=_cp(),
    )(e_flat, o_flat, w_taps, b.reshape(1, c_out).astype(jnp.float32), mask)
    return out.reshape(B, (oh + 4) * wp, c_out), wp


def _l5_body(x_ref, w_ref, o_ref, *, wp, mo):
    h = None
    for kh in range(4):
        for kw in range(4):
            t = kh * 4 + kw
            d = jnp.dot(x_ref[0, pl.ds(kh * wp + kw, mo), :], w_ref[t],
                        preferred_element_type=jnp.float32)
            h = d if h is None else h + d
    o_ref[0] = jax.nn.sigmoid(h)


def _layer5(flat, w5, hh, ww, wp):
    B, L, C = flat.shape
    mo = hh * wp
    wt = jnp.transpose(w5, (2, 3, 1, 0)).astype(_BF16)
    w_taps = jnp.stack([jnp.pad(wt[kh, kw], ((0, 0), (0, 7)))
                        for kh in range(4) for kw in range(4)])
    body = functools.partial(_l5_body, wp=wp, mo=mo)
    out = pl.pallas_call(
        body,
        out_shape=jax.ShapeDtypeStruct((B, mo, 8), jnp.float32),
        grid=(B,),
        in_specs=[
            pl.BlockSpec((1, L, C), lambda b: (b, 0, 0)),
            pl.BlockSpec((16, C, 8), lambda b: (0, 0, 0)),
        ],
        out_specs=pl.BlockSpec((1, mo, 8), lambda b: (b, 0, 0)),
        compiler_params=_cp(),
    )(flat, w_taps)
    return out[:, :, 0].reshape(B, hh, wp)[:, :, :ww].reshape(B, 1, hh, ww)


def kernel(w1, b1, w2, b2, w3, b3, w4, b4, w5, img_A, img_B):
    B, _, H, W = img_A.shape
    oh2, ow2 = H // 4, W // 4
    oh3, ow3 = H // 8, W // 8
    oh4, ow4 = H // 16, W // 16
    x = jnp.concatenate([img_A, img_B], axis=1).astype(_BF16)
    x = jnp.transpose(x, (0, 2, 3, 1))

    eo2, mw2 = _layer1(x, w1, b1)
    e2, o2, mw3 = _layer2(eo2, w2, b2, oh2, ow2, mw2)
    e3, o3, mw4 = _layer3(e2, o2, w3, b3, oh3, ow3, mw3)
    x5, wp = _layer4(e3, o3, w4, b4, oh4, ow4, mw4)
    return _layer5(x5, w5, oh4, ow4, wp)


# speedup vs baseline: 1.0807x; 1.0807x over previous
import functools

import jax
import jax.numpy as jnp
from jax import lax
from jax.experimental import pallas as pl
from jax.experimental.pallas import tpu as pltpu

_BF16 = jnp.bfloat16
_EPS = 1e-5
_VMEM = 48 * 1024 * 1024


def _cp():
    return pltpu.CompilerParams(dimension_semantics=("parallel",),
                                vmem_limit_bytes=_VMEM)


def _grid(B):
    return (B,)


def _bmap(B, extra):
    def m(b):
        return (b,) + (0,) * extra
    return m


def _cmap(extra):
    def m(b):
        return (0,) * extra
    return m


def _r8(n):
    return (n + 7) // 8 * 8


def _tap_weights_merged(w):
    c_out, c_in = w.shape[0], w.shape[1]
    wt = jnp.transpose(w, (2, 3, 1, 0)).astype(_BF16)
    taps = [wt[2 * dh:2 * dh + 2, 2 * dw:2 * dw + 2].reshape(4 * c_in, c_out)
            for dh in (0, 1) for dw in (0, 1)]
    return jnp.stack(taps)


def _tap_weights_split(w):
    c_out, c_in = w.shape[0], w.shape[1]
    wt = jnp.transpose(w, (2, 3, 1, 0)).astype(_BF16)
    taps = [wt[2 * dh + p, 2 * dw:2 * dw + 2].reshape(2 * c_in, c_out)
            for p in (0, 1) for dh in (0, 1) for dw in (0, 1)]
    return jnp.stack(taps)


def _stat_mask(mo, mw, ow, c):
    valid = (jnp.arange(mo, dtype=jnp.int32) % mw) < ow
    return jnp.broadcast_to(valid[:, None], (mo, c)).astype(jnp.float32)


def _norm_leaky(h, b_ref, m_ref, nvalid):
    h = h + b_ref[...]
    hm = h * m_ref[...]
    inv_n = 1.0 / nvalid
    mu = jnp.sum(hm, axis=0, keepdims=True) * inv_n
    var = jnp.sum(hm * hm, axis=0, keepdims=True) * inv_n - mu * mu
    h = (h - mu) * lax.rsqrt(var + _EPS)
    return jnp.maximum(h, 0.2 * h)


def _store_parity_merged(hb, oe_ref, oo_ref, oh, mw, ow, c, w2m):
    hf = oh // 2
    wf = ow // 2
    v = hb.reshape(hf, 2, mw, c)[:, :, :ow, :].reshape(hf, 2, wf, 2, c)
    oe_ref[0] = jnp.zeros((hf + 2, w2m, 2 * c), _BF16)
    oo_ref[0] = jnp.zeros((hf + 2, w2m, 2 * c), _BF16)
    oe_ref[0, 1:hf + 1, 1:wf + 1, 0:c] = v[:, 1, :, 1, :]
    oe_ref[0, 1:hf + 1, 0:wf, c:2 * c] = v[:, 1, :, 0, :]
    oo_ref[0, 0:hf, 1:wf + 1, 0:c] = v[:, 0, :, 1, :]
    oo_ref[0, 0:hf, 0:wf, c:2 * c] = v[:, 0, :, 0, :]


def _l1_body(p_ref, w_ref, b_ref, eo_ref, *, oh, ow, c, w2m):
    h = jnp.dot(p_ref[0], w_ref[...], preferred_element_type=jnp.float32)
    h = h + b_ref[...]
    h = jnp.maximum(h, 0.2 * h)
    hb = h.astype(_BF16)
    hf = oh // 2
    wf = ow // 2
    v = hb.reshape(hf, 2, wf, 2, c)
    eo_ref[0] = jnp.zeros((hf + 2, w2m, 4 * c), _BF16)
    eo_ref[0, 1:hf + 1, 1:wf + 1, 0:c] = v[:, 1, :, 1, :]
    eo_ref[0, 1:hf + 1, 0:wf, c:2 * c] = v[:, 1, :, 0, :]
    eo_ref[0, 0:hf, 1:wf + 1, 2 * c:3 * c] = v[:, 0, :, 1, :]
    eo_ref[0, 0:hf, 0:wf, 3 * c:4 * c] = v[:, 0, :, 0, :]


def _layer1(x, w1, b1):
    B, H, W, C = x.shape
    oh, ow = H // 2, W // 2
    c_out = w1.shape[0]
    xp = jnp.pad(x, ((0, 0), (1, 1), (1, 1), (0, 0)))
    s = xp.reshape(B, oh + 1, 2, ow + 1, 2, C)
    s = s.transpose(0, 1, 3, 2, 4, 5).reshape(B, oh + 1, ow + 1, 4 * C)
    p = jnp.concatenate(
        [s[:, dh:dh + oh, dw:dw + ow] for dh in (0, 1) for dw in (0, 1)],
        axis=-1).reshape(B, oh * ow, 16 * C)
    wt = jnp.transpose(w1, (2, 3, 1, 0)).astype(_BF16)
    w2d = wt.reshape(2, 2, 2, 2, C, c_out).transpose(0, 2, 1, 3, 4, 5)
    w2d = w2d.reshape(16 * C, c_out)
    rr = oh // 2 + 2
    w2m = _r8(ow // 2 + 1)
    body = functools.partial(_l1_body, oh=oh, ow=ow, c=c_out, w2m=w2m)
    eo = pl.pallas_call(
        body,
        out_shape=jax.ShapeDtypeStruct((B, rr, w2m, 4 * c_out), _BF16),
        grid=_grid(B),
        in_specs=[
            pl.BlockSpec((1, oh * ow, 16 * C), _bmap(B, 2)),
            pl.BlockSpec((16 * C, c_out), _cmap(2)),
            pl.BlockSpec((1, c_out), _cmap(2)),
        ],
        out_specs=pl.BlockSpec((1, rr, w2m, 4 * c_out), _bmap(B, 3)),
        compiler_params=_cp(),
    )(p, w2d, b1.reshape(1, c_out).astype(jnp.float32))
    return eo.reshape(B, rr * w2m, 4 * c_out), w2m


def _l2_body(eo_ref, w_ref, b_ref, m_ref, oe_ref, oo_ref, *, mw, mo, oh, ow,
             nvalid, c, w2m):
    shifts = (0, 1, mw, mw + 1)
    h = jnp.dot(eo_ref[0, pl.ds(0, mo), :], w_ref[0],
                preferred_element_type=jnp.float32)
    for t in range(1, 4):
        h = h + jnp.dot(eo_ref[0, pl.ds(shifts[t], mo), :], w_ref[t],
                        preferred_element_type=jnp.float32)
    h = _norm_leaky(h, b_ref, m_ref, nvalid)
    _store_parity_merged(h.astype(_BF16), oe_ref, oo_ref, oh, mw, ow, c, w2m)


def _layer2(eo_flat, w, b, oh, ow, mw):
    B, L, k4 = eo_flat.shape
    c_out = w.shape[0]
    mo = oh * mw
    w_taps = _tap_weights_merged(w)
    mask = _stat_mask(mo, mw, ow, c_out)
    rr = oh // 2 + 2
    w2m = _r8(ow // 2 + 1)
    body = functools.partial(_l2_body, mw=mw, mo=mo, oh=oh, ow=ow,
                             nvalid=oh * ow, c=c_out, w2m=w2m)
    osd = jax.ShapeDtypeStruct((B, rr, w2m, 2 * c_out), _BF16)
    obs = pl.BlockSpec((1, rr, w2m, 2 * c_out), _bmap(B, 3))
    e, o = pl.pallas_call(
        body,
        out_shape=(osd, osd),
        grid=_grid(B),
        in_specs=[
            pl.BlockSpec((1, L, k4), _bmap(B, 2)),
            pl.BlockSpec((4, k4, c_out), _cmap(3)),
            pl.BlockSpec((1, c_out), _cmap(2)),
            pl.BlockSpec((mo, c_out), _cmap(2)),
        ],
        out_specs=(obs, obs),
        compiler_params=_cp(),
    )(eo_flat, w_taps, b.reshape(1, c_out).astype(jnp.float32), mask)
    return (e.reshape(B, rr * w2m, 2 * c_out),
            o.reshape(B, rr * w2m, 2 * c_out), w2m)


def _l3_body(e_ref, o_ref, w_ref, b_ref, m_ref, oe_ref, oo_ref, *, mw, mo,
             oh, ow, nvalid, c, w2m):
    h = None
    for i, (dh, dw) in enumerate(((0, 0), (0, 1), (1, 0), (1, 1))):
        s = dh * mw + dw
        d = jnp.dot(e_ref[0, pl.ds(s, mo), :], w_ref[i],
                    preferred_element_type=jnp.float32)
        h = d if h is None else h + d
        h = h + jnp.dot(o_ref[0, pl.ds(s, mo), :], w_ref[4 + i],
                        preferred_element_type=jnp.float32)
    h = _norm_leaky(h, b_ref, m_ref, nvalid)
    _store_parity_merged(h.astype(_BF16), oe_ref, oo_ref, oh, mw, ow, c, w2m)


def _layer3(e_flat, o_flat, w, b, oh, ow, mw):
    B, L, k2 = e_flat.shape
    c_out = w.shape[0]
    mo = oh * mw
    w_taps = _tap_weights_split(w)
    mask = _stat_mask(mo, mw, ow, c_out)
    rr = oh // 2 + 2
    w2m = _r8(ow // 2 + 1)
    body = functools.partial(_l3_body, mw=mw, mo=mo, oh=oh, ow=ow,
                             nvalid=oh * ow, c=c_out, w2m=w2m)
    osd = jax.ShapeDtypeStruct((B, rr, w2m, 2 * c_out), _BF16)
    obs = pl.BlockSpec((1, rr, w2m, 2 * c_out), _bmap(B, 3))
    ibs = pl.BlockSpec((1, L, k2), _bmap(B, 2))
    e, o = pl.pallas_call(
        body,
        out_shape=(osd, osd),
        grid=_grid(B),
        in_specs=[
            ibs, ibs,
            pl.BlockSpec((8, k2, c_out), _cmap(3)),
            pl.BlockSpec((1, c_out), _cmap(2)),
            pl.BlockSpec((mo, c_out), _cmap(2)),
        ],
        out_specs=(obs, obs),
        compiler_params=_cp(),
    )(e_flat, o_flat, w_taps, b.reshape(1, c_out).astype(jnp.float32), mask)
    return (e.reshape(B, rr * w2m, 2 * c_out),
            o.reshape(B, rr * w2m, 2 * c_out), w2m)


def _l4_body(e_ref, o_ref, w_ref, b_ref, m_ref, o5_ref, *, mw, mo, oh, ow,
             nvalid, c, wp):
    h = None
    for i, (dh, dw) in enumerate(((0, 0), (0, 1), (1, 0), (1, 1))):
        s = dh * mw + dw
        d = jnp.dot(e_ref[0, pl.ds(s, mo), :], w_ref[i],
                    preferred_element_type=jnp.float32)
        h = d if h is None else h + d
        h = h + jnp.dot(o_ref[0, pl.ds(s, mo), :], w_ref[4 + i],
                        preferred_element_type=jnp.float32)
    h = _norm_leaky(h, b_ref, m_ref, nvalid)
    hb = h.astype(_BF16).reshape(oh, mw, c)[:, :ow, :]
    o5_ref[0] = jnp.zeros((oh + 4, wp, c), _BF16)
    o5_ref[0, 2:oh + 2, 2:ow + 2, :] = hb


def _layer4(e_flat, o_flat, w, b, oh, ow, mw):
    B, L, k2 = e_flat.shape
    c_out = w.shape[0]
    mo = oh * mw
    w_taps = _tap_weights_split(w)
    mask = _stat_mask(mo, mw, ow, c_out)
    wp = _r8(ow + 3)
    body = functools.partial(_l4_body, mw=mw, mo=mo, oh=oh, ow=ow,
                             nvalid=oh * ow, c=c_out, wp=wp)
    osd = jax.ShapeDtypeStruct((B, oh + 4, wp, c_out), _BF16)
    obs = pl.BlockSpec((1, oh + 4, wp, c_out), _bmap(B, 3))
    ibs = pl.BlockSpec((1, L, k2), _bmap(B, 2))
    out = pl.pallas_call(
        body,
        out_shape=osd,
        grid=_grid(B),
        in_specs=[
            ibs, ibs,
            pl.BlockSpec((8, k2, c_out), _cmap(3)),
            pl.BlockSpec((1, c_out), _cmap(2)),
            pl.BlockSpec((mo, c_out), _cmap(2)),
        ],
        out_specs=obs,
        compiler_params=_cp(),
    )(e_flat, o_flat, w_taps, b.reshape(1, c_out).astype(jnp.float32), mask)
    return out.reshape(B, (oh + 4) * wp, c_out), wp


def _l5_body(x_ref, w_ref, o_ref, *, wp, mo):
    h = None
    for kh in range(4):
        for kw in range(4):
            t = kh * 4 + kw
            d = jnp.dot(x_ref[0, pl.ds(kh * wp + kw, mo), :], w_ref[t],
                        preferred_element_type=jnp.float32)
            h = d if h is None else h + d
    o_ref[0] = jax.nn.sigmoid(h)


def _layer5(flat, w5, hh, ww, wp):
    B, L, C = flat.shape
    mo = hh * wp
    wt = jnp.transpose(w5, (2, 3, 1, 0)).astype(_BF16)
    w_taps = jnp.stack([jnp.pad(wt[kh, kw], ((0, 0), (0, 7)))
                        for kh in range(4) for kw in range(4)])
    body = functools.partial(_l5_body, wp=wp, mo=mo)
    out = pl.pallas_call(
        body,
        out_shape=jax.ShapeDtypeStruct((B, mo, 8), jnp.float32),
        grid=_grid(B),
        in_specs=[
            pl.BlockSpec((1, L, C), _bmap(B, 2)),
            pl.BlockSpec((16, C, 8), _cmap(3)),
        ],
        out_specs=pl.BlockSpec((1, mo, 8), _bmap(B, 2)),
        compiler_params=_cp(),
    )(flat, w_taps)
    return out[:, :, 0].reshape(B, hh, wp)[:, :, :ww].reshape(B, 1, hh, ww)


def kernel(w1, b1, w2, b2, w3, b3, w4, b4, w5, img_A, img_B):
    B, _, H, W = img_A.shape
    oh2, ow2 = H // 4, W // 4
    oh3, ow3 = H // 8, W // 8
    oh4, ow4 = H // 16, W // 16
    x = jnp.concatenate([img_A, img_B], axis=1).astype(_BF16)
    x = jnp.transpose(x, (0, 2, 3, 1))

    eo2, mw2 = _layer1(x, w1, b1)
    e2, o2, mw3 = _layer2(eo2, w2, b2, oh2, ow2, mw2)
    e3, o3, mw4 = _layer3(e2, o2, w3, b3, oh3, ow3, mw3)
    x5, wp = _layer4(e3, o3, w4, b4, oh4, ow4, mw4)
    return _layer5(x5, w5, oh4, ow4, wp)


# speedup vs baseline: 1.1153x; 1.0321x over previous
import functools

import jax
import jax.numpy as jnp
from jax import lax
from jax.experimental import pallas as pl
from jax.experimental.pallas import tpu as pltpu

_BF16 = jnp.bfloat16
_EPS = 1e-5
_VMEM = 48 * 1024 * 1024


def _cp():
    return pltpu.CompilerParams(dimension_semantics=("parallel",),
                                vmem_limit_bytes=_VMEM)


def _grid(B):
    return (B,)


def _bmap(B, extra):
    def m(b):
        return (b,) + (0,) * extra
    return m


def _cmap(extra):
    def m(b):
        return (0,) * extra
    return m


def _r8(n):
    return (n + 7) // 8 * 8


def _tap_weights_merged(w):
    c_out, c_in = w.shape[0], w.shape[1]
    wt = jnp.transpose(w, (2, 3, 1, 0)).astype(_BF16)
    taps = [wt[2 * dh:2 * dh + 2, 2 * dw:2 * dw + 2].reshape(4 * c_in, c_out)
            for dh in (0, 1) for dw in (0, 1)]
    return jnp.stack(taps)


def _tap_weights_split(w):
    c_out, c_in = w.shape[0], w.shape[1]
    wt = jnp.transpose(w, (2, 3, 1, 0)).astype(_BF16)
    taps = [wt[2 * dh + p, 2 * dw:2 * dw + 2].reshape(2 * c_in, c_out)
            for p in (0, 1) for dh in (0, 1) for dw in (0, 1)]
    return jnp.stack(taps)


def _stat_mask(mo, mw, ow, c):
    valid = (jnp.arange(mo, dtype=jnp.int32) % mw) < ow
    return jnp.broadcast_to(valid[:, None], (mo, c)).astype(jnp.float32)


def _norm_leaky(h, b_ref, m_ref, nvalid):
    h = h + b_ref[...]
    hm = h * m_ref[...]
    inv_n = 1.0 / nvalid
    mu = jnp.sum(hm, axis=0, keepdims=True) * inv_n
    var = jnp.sum(hm * hm, axis=0, keepdims=True) * inv_n - mu * mu
    h = (h - mu) * lax.rsqrt(var + _EPS)
    return jnp.maximum(h, 0.2 * h)


def _store_parity_merged(hb, oe_ref, oo_ref, oh, mw, ow, c, w2m):
    hf = oh // 2
    wf = ow // 2
    v = hb.reshape(hf, 2, mw, c)[:, :, :ow, :].reshape(hf, 2, wf, 2, c)
    oe_ref[0] = jnp.zeros((hf + 2, w2m, 2 * c), _BF16)
    oo_ref[0] = jnp.zeros((hf + 2, w2m, 2 * c), _BF16)
    oe_ref[0, 1:hf + 1, 1:wf + 1, 0:c] = v[:, 1, :, 1, :].astype(_BF16)
    oe_ref[0, 1:hf + 1, 0:wf, c:2 * c] = v[:, 1, :, 0, :].astype(_BF16)
    oo_ref[0, 0:hf, 1:wf + 1, 0:c] = v[:, 0, :, 1, :].astype(_BF16)
    oo_ref[0, 0:hf, 0:wf, c:2 * c] = v[:, 0, :, 0, :].astype(_BF16)


def _l1_body(p_ref, w_ref, b_ref, eo_ref, *, oh, ow, c, w2m):
    h = jnp.dot(p_ref[0], w_ref[...], preferred_element_type=jnp.float32)
    h = h + b_ref[...]
    h = jnp.maximum(h, 0.2 * h)
    hf = oh // 2
    wf = ow // 2
    v = h.reshape(hf, 2, wf, 2, c)
    eo_ref[0] = jnp.zeros((hf + 2, w2m, 4 * c), _BF16)
    eo_ref[0, 1:hf + 1, 1:wf + 1, 0:c] = v[:, 1, :, 1, :].astype(_BF16)
    eo_ref[0, 1:hf + 1, 0:wf, c:2 * c] = v[:, 1, :, 0, :].astype(_BF16)
    eo_ref[0, 0:hf, 1:wf + 1, 2 * c:3 * c] = v[:, 0, :, 1, :].astype(_BF16)
    eo_ref[0, 0:hf, 0:wf, 3 * c:4 * c] = v[:, 0, :, 0, :].astype(_BF16)


def _layer1(x, w1, b1):
    B, H, W, C = x.shape
    oh, ow = H // 2, W // 2
    c_out = w1.shape[0]
    xp = jnp.pad(x, ((0, 0), (1, 1), (1, 1), (0, 0)))
    s = xp.reshape(B, oh + 1, 2, ow + 1, 2, C)
    s = s.transpose(0, 1, 3, 2, 4, 5).reshape(B, oh + 1, ow + 1, 4 * C)
    p = jnp.concatenate(
        [s[:, dh:dh + oh, dw:dw + ow] for dh in (0, 1) for dw in (0, 1)],
        axis=-1).reshape(B, oh * ow, 16 * C)
    wt = jnp.transpose(w1, (2, 3, 1, 0)).astype(_BF16)
    w2d = wt.reshape(2, 2, 2, 2, C, c_out).transpose(0, 2, 1, 3, 4, 5)
    w2d = w2d.reshape(16 * C, c_out)
    rr = oh // 2 + 2
    w2m = _r8(ow // 2 + 1)
    body = functools.partial(_l1_body, oh=oh, ow=ow, c=c_out, w2m=w2m)
    eo = pl.pallas_call(
        body,
        out_shape=jax.ShapeDtypeStruct((B, rr, w2m, 4 * c_out), _BF16),
        grid=_grid(B),
        in_specs=[
            pl.BlockSpec((1, oh * ow, 16 * C), _bmap(B, 2)),
            pl.BlockSpec((16 * C, c_out), _cmap(2)),
            pl.BlockSpec((1, c_out), _cmap(2)),
        ],
        out_specs=pl.BlockSpec((1, rr, w2m, 4 * c_out), _bmap(B, 3)),
        compiler_params=_cp(),
    )(p, w2d, b1.reshape(1, c_out).astype(jnp.float32))
    return eo.reshape(B, rr * w2m, 4 * c_out), w2m


def _l2_body(eo_ref, w_ref, b_ref, m_ref, oe_ref, oo_ref, *, mw, mo, oh, ow,
             nvalid, c, w2m):
    shifts = (0, 1, mw, mw + 1)
    h = jnp.dot(eo_ref[0, pl.ds(0, mo), :], w_ref[0],
                preferred_element_type=jnp.float32)
    for t in range(1, 4):
        h = h + jnp.dot(eo_ref[0, pl.ds(shifts[t], mo), :], w_ref[t],
                        preferred_element_type=jnp.float32)
    h = _norm_leaky(h, b_ref, m_ref, nvalid)
    _store_parity_merged(h, oe_ref, oo_ref, oh, mw, ow, c, w2m)


def _layer2(eo_flat, w, b, oh, ow, mw):
    B, L, k4 = eo_flat.shape
    c_out = w.shape[0]
    mo = oh * mw
    w_taps = _tap_weights_merged(w)
    mask = _stat_mask(mo, mw, ow, c_out)
    rr = oh // 2 + 2
    w2m = _r8(ow // 2 + 1)
    body = functools.partial(_l2_body, mw=mw, mo=mo, oh=oh, ow=ow,
                             nvalid=oh * ow, c=c_out, w2m=w2m)
    osd = jax.ShapeDtypeStruct((B, rr, w2m, 2 * c_out), _BF16)
    obs = pl.BlockSpec((1, rr, w2m, 2 * c_out), _bmap(B, 3))
    e, o = pl.pallas_call(
        body,
        out_shape=(osd, osd),
        grid=_grid(B),
        in_specs=[
            pl.BlockSpec((1, L, k4), _bmap(B, 2)),
            pl.BlockSpec((4, k4, c_out), _cmap(3)),
            pl.BlockSpec((1, c_out), _cmap(2)),
            pl.BlockSpec((mo, c_out), _cmap(2)),
        ],
        out_specs=(obs, obs),
        compiler_params=_cp(),
    )(eo_flat, w_taps, b.reshape(1, c_out).astype(jnp.float32), mask)
    return (e.reshape(B, rr * w2m, 2 * c_out),
            o.reshape(B, rr * w2m, 2 * c_out), w2m)


def _l3_body(e_ref, o_ref, w_ref, b_ref, m_ref, oe_ref, oo_ref, *, mw, mo,
             oh, ow, nvalid, c, w2m):
    h = None
    for i, (dh, dw) in enumerate(((0, 0), (0, 1), (1, 0), (1, 1))):
        s = dh * mw + dw
        d = jnp.dot(e_ref[0, pl.ds(s, mo), :], w_ref[i],
                    preferred_element_type=jnp.float32)
        h = d if h is None else h + d
        h = h + jnp.dot(o_ref[0, pl.ds(s, mo), :], w_ref[4 + i],
                        preferred_element_type=jnp.float32)
    h = _norm_leaky(h, b_ref, m_ref, nvalid)
    _store_parity_merged(h, oe_ref, oo_ref, oh, mw, ow, c, w2m)


def _layer3(e_flat, o_flat, w, b, oh, ow, mw):
    B, L, k2 = e_flat.shape
    c_out = w.shape[0]
    mo = oh * mw
    w_taps = _tap_weights_split(w)
    mask = _stat_mask(mo, mw, ow, c_out)
    rr = oh // 2 + 2
    w2m = _r8(ow // 2 + 1)
    body = functools.partial(_l3_body, mw=mw, mo=mo, oh=oh, ow=ow,
                             nvalid=oh * ow, c=c_out, w2m=w2m)
    osd = jax.ShapeDtypeStruct((B, rr, w2m, 2 * c_out), _BF16)
    obs = pl.BlockSpec((1, rr, w2m, 2 * c_out), _bmap(B, 3))
    ibs = pl.BlockSpec((1, L, k2), _bmap(B, 2))
    e, o = pl.pallas_call(
        body,
        out_shape=(osd, osd),
        grid=_grid(B),
        in_specs=[
            ibs, ibs,
            pl.BlockSpec((8, k2, c_out), _cmap(3)),
            pl.BlockSpec((1, c_out), _cmap(2)),
            pl.BlockSpec((mo, c_out), _cmap(2)),
        ],
        out_specs=(obs, obs),
        compiler_params=_cp(),
    )(e_flat, o_flat, w_taps, b.reshape(1, c_out).astype(jnp.float32), mask)
    return (e.reshape(B, rr * w2m, 2 * c_out),
            o.reshape(B, rr * w2m, 2 * c_out), w2m)


def _l4_body(e_ref, o_ref, w_ref, b_ref, m_ref, o5_ref, *, mw, mo, oh, ow,
             nvalid, c, wp):
    h = None
    for i, (dh, dw) in enumerate(((0, 0), (0, 1), (1, 0), (1, 1))):
        s = dh * mw + dw
        d = jnp.dot(e_ref[0, pl.ds(s, mo), :], w_ref[i],
                    preferred_element_type=jnp.float32)
        h = d if h is None else h + d
        h = h + jnp.dot(o_ref[0, pl.ds(s, mo), :], w_ref[4 + i],
                        preferred_element_type=jnp.float32)
    h = _norm_leaky(h, b_ref, m_ref, nvalid)
    hb = h.astype(_BF16).reshape(oh, mw, c)[:, :ow, :]
    o5_ref[0] = jnp.zeros((oh + 4, wp, c), _BF16)
    o5_ref[0, 2:oh + 2, 2:ow + 2, :] = hb


def _layer4(e_flat, o_flat, w, b, oh, ow, mw):
    B, L, k2 = e_flat.shape
    c_out = w.shape[0]
    mo = oh * mw
    w_taps = _tap_weights_split(w)
    mask = _stat_mask(mo, mw, ow, c_out)
    wp = _r8(ow + 3)
    body = functools.partial(_l4_body, mw=mw, mo=mo, oh=oh, ow=ow,
                             nvalid=oh * ow, c=c_out, wp=wp)
    osd = jax.ShapeDtypeStruct((B, oh + 4, wp, c_out), _BF16)
    obs = pl.BlockSpec((1, oh + 4, wp, c_out), _bmap(B, 3))
    ibs = pl.BlockSpec((1, L, k2), _bmap(B, 2))
    out = pl.pallas_call(
        body,
        out_shape=osd,
        grid=_grid(B),
        in_specs=[
            ibs, ibs,
            pl.BlockSpec((8, k2, c_out), _cmap(3)),
            pl.BlockSpec((1, c_out), _cmap(2)),
            pl.BlockSpec((mo, c_out), _cmap(2)),
        ],
        out_specs=obs,
        compiler_params=_cp(),
    )(e_flat, o_flat, w_taps, b.reshape(1, c_out).astype(jnp.float32), mask)
    return out.reshape(B, (oh + 4) * wp, c_out), wp


def _l5_body(x_ref, w_ref, o_ref, *, wp, mo):
    h = None
    for kh in range(4):
        for kw in range(4):
            t = kh * 4 + kw
            d = jnp.dot(x_ref[0, pl.ds(kh * wp + kw, mo), :], w_ref[t],
                        preferred_element_type=jnp.float32)
            h = d if h is None else h + d
    o_ref[0] = jax.nn.sigmoid(h)


def _layer5(flat, w5, hh, ww, wp):
    B, L, C = flat.shape
    mo = hh * wp
    wt = jnp.transpose(w5, (2, 3, 1, 0)).astype(_BF16)
    w_taps = jnp.stack([jnp.pad(wt[kh, kw], ((0, 0), (0, 7)))
                        for kh in range(4) for kw in range(4)])
    body = functools.partial(_l5_body, wp=wp, mo=mo)
    out = pl.pallas_call(
        body,
        out_shape=jax.ShapeDtypeStruct((B, mo, 8), jnp.float32),
        grid=_grid(B),
        in_specs=[
            pl.BlockSpec((1, L, C), _bmap(B, 2)),
            pl.BlockSpec((16, C, 8), _cmap(3)),
        ],
        out_specs=pl.BlockSpec((1, mo, 8), _bmap(B, 2)),
        compiler_params=_cp(),
    )(flat, w_taps)
    return out[:, :, 0].reshape(B, hh, wp)[:, :, :ww].reshape(B, 1, hh, ww)


def kernel(w1, b1, w2, b2, w3, b3, w4, b4, w5, img_A, img_B):
    B, _, H, W = img_A.shape
    oh2, ow2 = H // 4, W // 4
    oh3, ow3 = H // 8, W // 8
    oh4, ow4 = H // 16, W // 16
    x = jnp.concatenate([img_A, img_B], axis=1).astype(_BF16)
    x = jnp.transpose(x, (0, 2, 3, 1))

    eo2, mw2 = _layer1(x, w1, b1)
    e2, o2, mw3 = _layer2(eo2, w2, b2, oh2, ow2, mw2)
    e3, o3, mw4 = _layer3(e2, o2, w3, b3, oh3, ow3, mw3)
    x5, wp = _layer4(e3, o3, w4, b4, oh4, ow4, mw4)
    return _layer5(x5, w5, oh4, ow4, wp)
